# Initial kernel scaffold; baseline (speedup 1.0000x reference)
import jax
import jax.numpy as jnp
from jax.experimental import pallas as pl


def kernel(tokenized_prompts, token_embedding, ctx):
    raise NotImplementedError("write your pallas kernel here")


# speedup vs baseline: 2.9375x; 2.9375x over previous
import functools

import jax
import jax.numpy as jnp
from jax import lax
from jax.experimental import pallas as pl
from jax.experimental.pallas import tpu as pltpu
from jax.experimental.pallas import tpu_sc as plsc

CTX_DIM = 768
N_CLS = 1000
SEQ = 77
N_CTX = 4
LANES = 16


def _sc_prompts(idx80, token_embedding, ctx):
    info = plsc.get_sparse_core_info()
    nw = info.num_cores * info.num_subcores
    base = N_CLS // nw
    extra = N_CLS - base * nw
    mesh = plsc.VectorSubcoreMesh(core_axis_name="c", subcore_axis_name="s")

    @functools.partial(
        pl.kernel,
        mesh=mesh,
        out_type=jax.ShapeDtypeStruct((N_CLS, SEQ, CTX_DIM), jnp.float32),
        scratch_types=[
            pltpu.VMEM((1, SEQ), jnp.int32),
            pltpu.VMEM((N_CTX, CTX_DIM), jnp.float32),
            pltpu.VMEM((SEQ, CTX_DIM), jnp.float32),
            pltpu.SemaphoreType.DMA,
        ],
    )
    def k(idx_hbm, table_hbm, ctx_hbm, out_hbm, idx_v, ctx_v, buf_v, sem):
        wid = lax.axis_index("s") * info.num_cores + lax.axis_index("c")
        start = wid * base + jnp.minimum(wid, extra)
        count = base + jnp.where(wid < extra, 1, 0)

        pltpu.sync_copy(ctx_hbm, ctx_v)

        def body(i, carry):
            c = start + i
            pltpu.sync_copy(idx_hbm.at[c], idx_v)
            pltpu.async_copy(
                table_hbm.at[idx_v.at[0]], buf_v, sem).wait()
            for r in range(N_CTX):
                for j in range(CTX_DIM // LANES):
                    buf_v[1 + r, pl.ds(j * LANES, LANES)] = (
                        ctx_v[r, pl.ds(j * LANES, LANES)])
            pltpu.sync_copy(buf_v, out_hbm.at[c])
            return carry

        lax.fori_loop(0, count, body, 0)

    return k(idx80, token_embedding, ctx)


def kernel(tokenized_prompts, token_embedding, ctx):
    z = jnp.zeros((N_CLS, N_CTX), jnp.int32)
    idx3 = jnp.concatenate(
        [tokenized_prompts[:, :1], z, tokenized_prompts[:, 1 + N_CTX:]],
        axis=1).reshape(N_CLS, 1, SEQ)
    return _sc_prompts(idx3, token_embedding, ctx)
